# Initial kernel scaffold; baseline (speedup 1.0000x reference)
import jax
import jax.numpy as jnp
from jax.experimental import pallas as pl


def kernel(x, W_emb, W_q, pos_enc):
    raise NotImplementedError("write your pallas kernel here")


# speedup vs baseline: 4.1834x; 4.1834x over previous
import functools

import jax
import jax.numpy as jnp
from jax import lax
from jax.experimental import pallas as pl
from jax.experimental.pallas import tpu as pltpu
from jax.experimental.pallas import tpu_sc as plsc

B, N, L, D = 1, 512, 1024, 64
V = 21
TOK = B * N * L
NC, NS = 2, 16
NW = NC * NS
CHUNK = 1024
CH_PER_W = TOK // (NW * CHUNK)
IDX_ROWS = CHUNK // 128


def _table_body(we_ref, wq_ref, pe_ref, out_ref):
    pe = pe_ref[...]
    we = we_ref[...]
    for q in range(2):
        wq = wq_ref[q]
        out_ref[q] = pe[:, None, :] + we[None, :, :] + wq[None, None, :]


def _build_table(W_emb, W_q, pos_enc):
    t = pl.pallas_call(
        _table_body,
        out_shape=jax.ShapeDtypeStruct((2, L, V, D), jnp.float32),
    )(W_emb, W_q, pos_enc)
    return t.reshape(2 * L * V, D)


def _gather_kernel(table_hbm, x_hbm, out_hbm, x_v, idx_v, rows_v, sem):
    wid = lax.axis_index("s") * NC + lax.axis_index("c")

    def chunk_body(c, carry):
        g = wid * CH_PER_W + c
        base = g * CHUNK
        pltpu.sync_copy(x_hbm.at[pl.ds(base, CHUNK)], x_v)
        qoff = jnp.where(g > 0, L * V, 0).astype(jnp.int32)

        for j in range(IDX_ROWS):
            def idx_body(i, _):
                t = j * 128 + i * 16
                xv = x_v[pl.ds(t, 16)]
                lv = lax.iota(jnp.int32, 16) + t
                idx_v[j, pl.ds(i * 16, 16)] = xv + lv * V + qoff
                return 0

            lax.fori_loop(0, 128 // 16, idx_body, 0)

        copies = [
            pltpu.async_copy(
                table_hbm.at[idx_v.at[j]],
                rows_v.at[pl.ds(j * 128, 128)],
                sem,
            )
            for j in range(IDX_ROWS)
        ]
        for cp in copies:
            cp.wait()
        pltpu.sync_copy(rows_v, out_hbm.at[pl.ds(base, CHUNK)])
        return carry

    lax.fori_loop(0, CH_PER_W, chunk_body, 0)


def _gather(table, x_flat):
    mesh = plsc.VectorSubcoreMesh(core_axis_name="c", subcore_axis_name="s")
    k = functools.partial(
        pl.kernel,
        mesh=mesh,
        out_type=jax.ShapeDtypeStruct((TOK, D), jnp.float32),
        scratch_types=[
            pltpu.VMEM((CHUNK,), jnp.int32),
            pltpu.VMEM((IDX_ROWS, 128), jnp.int32),
            pltpu.VMEM((CHUNK, D), jnp.float32),
            pltpu.SemaphoreType.DMA,
        ],
        compiler_params=pltpu.CompilerParams(use_tc_tiling_on_sc=False),
    )(_gather_kernel)
    return k(table, x_flat)


def kernel(x, W_emb, W_q, pos_enc):
    table = _build_table(W_emb, W_q, pos_enc)
    x_flat = x.reshape(TOK).astype(jnp.int32)
    out = _gather(table, x_flat)
    return out.reshape(B, N, L, D)


# speedup vs baseline: 4.3121x; 1.0308x over previous
import functools

import jax
import jax.numpy as jnp
from jax import lax
from jax.experimental import pallas as pl
from jax.experimental.pallas import tpu as pltpu
from jax.experimental.pallas import tpu_sc as plsc

B, N, L, D = 1, 512, 1024, 64
V = 21
TOK = B * N * L
NC, NS = 2, 16
NW = NC * NS
CHUNK = 512
CH_PER_W = TOK // (NW * CHUNK)
PAIRS = CH_PER_W // 2
IDX_ROWS = CHUNK // 128


def _table_body(we_ref, wq_ref, pe_ref, out_ref):
    pe = pe_ref[...]
    we = we_ref[...]
    for q in range(2):
        wq = wq_ref[q]
        out_ref[q] = pe[:, None, :] + we[None, :, :] + wq[None, None, :]


def _build_table(W_emb, W_q, pos_enc):
    t = pl.pallas_call(
        _table_body,
        out_shape=jax.ShapeDtypeStruct((2, L, V, D), jnp.float32),
    )(W_emb, W_q, pos_enc)
    return t.reshape(2 * L * V, D)


def _gather_kernel(
    table_hbm, x_hbm, out_hbm,
    x_v0, x_v1, idx_v0, idx_v1, rows_v0, rows_v1,
    gsem0, gsem1, ssem0, ssem1,
):
    wid = lax.axis_index("s") * NC + lax.axis_index("c")

    def prepare(g, x_v, idx_v, rows_v, gsem):
        base = g * CHUNK
        pltpu.sync_copy(x_hbm.at[pl.ds(base, CHUNK)], x_v)
        l_base = (g % (L // CHUNK)) * CHUNK
        qoff = jnp.where(g >= L // CHUNK, L * V, 0).astype(jnp.int32)

        for j in range(IDX_ROWS):
            def idx_body(i, _):
                t = j * 128 + i * 16
                xv = x_v[pl.ds(t, 16)]
                lv = lax.iota(jnp.int32, 16) + (l_base + t)
                idx_v[j, pl.ds(i * 16, 16)] = xv + lv * V + qoff
                return 0

            lax.fori_loop(0, 128 // 16, idx_body, 0)

        return [
            pltpu.async_copy(
                table_hbm.at[idx_v.at[j]],
                rows_v.at[pl.ds(j * 128, 128)],
                gsem,
            )
            for j in range(IDX_ROWS)
        ]

    def pair_body(p, carry):
        g0 = wid * CH_PER_W + 2 * p
        g1 = g0 + 1

        @pl.when(p > 0)
        def _():
            pltpu.make_async_copy(
                rows_v0, out_hbm.at[pl.ds((g0 - 2) * CHUNK, CHUNK)], ssem0
            ).wait()

        cps0 = prepare(g0, x_v0, idx_v0, rows_v0, gsem0)

        @pl.when(p > 0)
        def _():
            pltpu.make_async_copy(
                rows_v1, out_hbm.at[pl.ds((g1 - 2) * CHUNK, CHUNK)], ssem1
            ).wait()

        cps1 = prepare(g1, x_v1, idx_v1, rows_v1, gsem1)

        for cp in cps0:
            cp.wait()
        pltpu.async_copy(rows_v0, out_hbm.at[pl.ds(g0 * CHUNK, CHUNK)], ssem0)
        for cp in cps1:
            cp.wait()
        pltpu.async_copy(rows_v1, out_hbm.at[pl.ds(g1 * CHUNK, CHUNK)], ssem1)
        return carry

    lax.fori_loop(0, PAIRS, pair_body, 0)

    g_last0 = wid * CH_PER_W + CH_PER_W - 2
    g_last1 = wid * CH_PER_W + CH_PER_W - 1
    pltpu.make_async_copy(
        rows_v0, out_hbm.at[pl.ds(g_last0 * CHUNK, CHUNK)], ssem0
    ).wait()
    pltpu.make_async_copy(
        rows_v1, out_hbm.at[pl.ds(g_last1 * CHUNK, CHUNK)], ssem1
    ).wait()


def _gather(table, x_flat):
    mesh = plsc.VectorSubcoreMesh(core_axis_name="c", subcore_axis_name="s")
    k = functools.partial(
        pl.kernel,
        mesh=mesh,
        out_type=jax.ShapeDtypeStruct((TOK, D), jnp.float32),
        scratch_types=[
            pltpu.VMEM((CHUNK,), jnp.int32),
            pltpu.VMEM((CHUNK,), jnp.int32),
            pltpu.VMEM((IDX_ROWS, 128), jnp.int32),
            pltpu.VMEM((IDX_ROWS, 128), jnp.int32),
            pltpu.VMEM((CHUNK, D), jnp.float32),
            pltpu.VMEM((CHUNK, D), jnp.float32),
            pltpu.SemaphoreType.DMA,
            pltpu.SemaphoreType.DMA,
            pltpu.SemaphoreType.DMA,
            pltpu.SemaphoreType.DMA,
        ],
        compiler_params=pltpu.CompilerParams(use_tc_tiling_on_sc=False),
    )(_gather_kernel)
    return k(table, x_flat)


def kernel(x, W_emb, W_q, pos_enc):
    table = _build_table(W_emb, W_q, pos_enc)
    x_flat = x.reshape(TOK).astype(jnp.int32)
    out = _gather(table, x_flat)
    return out.reshape(B, N, L, D)
